# Initial kernel scaffold; baseline (speedup 1.0000x reference)
import jax
import jax.numpy as jnp
from jax.experimental import pallas as pl


def kernel(x, mask, edge_index, W1, b1, g1, be1, W2s, W2n, b2, g2, be2, W3, b3, g3, be3):
    raise NotImplementedError("write your pallas kernel here")


# speedup vs baseline: 14.6801x; 14.6801x over previous
import functools

import jax
import jax.numpy as jnp
from jax.experimental import pallas as pl
from jax.experimental.pallas import tpu as pltpu
from jax.experimental.pallas import tpu_sc as plsc

N = 10000
E = 320000
INPLANES = 128
PLANES = 32
EXPANSION = 4
EPS = 1e-5

PW = 48
NC = 2
NS = 16
NW = NC * NS
EPW = E // NW
K = 80
CH = EPW // K
ZR = N // NS

G = 10
R = N // G

_HIGH = jax.lax.Precision.HIGHEST


def _dot(a, b):
    return jnp.dot(a, b, preferred_element_type=jnp.float32, precision=_HIGH)


def _leaky(h):
    return jnp.where(h >= 0, h, 0.1 * h)


def _norm_from_stats(o, stats, g, b):
    mu = stats[0:1, :] * (1.0 / N)
    var = stats[1:2, :] * (1.0 / N) - mu * mu
    return g * (o - mu) / jnp.sqrt(var + EPS) + b


def _accum_stats(i, o, stats_ref):
    @pl.when(i == 0)
    def _():
        stats_ref[...] = jnp.zeros_like(stats_ref)
    s = jnp.sum(o, axis=0, keepdims=True)
    sq = jnp.sum(o * o, axis=0, keepdims=True)
    stats_ref[...] += jnp.concatenate([s, sq], axis=0)


def _rowspec(w):
    return pl.BlockSpec((R, w), lambda i: (i, 0))


def _fullspec(h, w):
    return pl.BlockSpec((h, w), lambda i: (0, 0))


def _stage_a1(x_ref, m_ref, w1_ref, b1_ref, o_ref, stats_ref):
    i = pl.program_id(0)
    m = m_ref[...]
    hold = (m > 0).astype(jnp.float32)
    ratio = hold / jnp.clip(m, EPS, None)
    o = _dot(x_ref[...] * m, w1_ref[...])
    o = o * ratio + b1_ref[...] * hold
    o_ref[...] = o
    _accum_stats(i, o, stats_ref)


_stage_a1_call = pl.pallas_call(
    _stage_a1,
    grid=(G,),
    in_specs=[_rowspec(INPLANES), _rowspec(1), _fullspec(INPLANES, PW),
              _fullspec(1, PW)],
    out_specs=(_rowspec(PW), _fullspec(2, PW)),
    out_shape=(
        jax.ShapeDtypeStruct((N, PW), jnp.float32),
        jax.ShapeDtypeStruct((2, PW), jnp.float32),
    ),
)


def _stage_a2(o_ref, m_ref, stats_ref, g1_ref, be1_ref, pay_ref):
    m = m_ref[...]
    hold = (m > 0).astype(jnp.float32)
    h = _leaky(_norm_from_stats(o_ref[...], stats_ref[...],
                                g1_ref[...], be1_ref[...]))
    hm = h * hold
    ci = jax.lax.broadcasted_iota(jnp.int32, (R, PW), 1)
    holdb = jnp.broadcast_to(hold, (R, PW))
    pay_ref[...] = jnp.where(ci < PLANES, hm,
                             jnp.where(ci == PLANES, holdb,
                                       jnp.where(ci == PLANES + 1, 1.0, 0.0)))


_stage_a2_call = pl.pallas_call(
    _stage_a2,
    grid=(G,),
    in_specs=[_rowspec(PW), _rowspec(1), _fullspec(2, PW), _fullspec(1, PW),
              _fullspec(1, PW)],
    out_specs=_rowspec(PW),
    out_shape=jax.ShapeDtypeStruct((N, PW), jnp.float32),
)


@functools.cache
def _sc_segsum_call():
    mesh = plsc.VectorSubcoreMesh(
        core_axis_name="c", subcore_axis_name="s",
        num_cores=NC, num_subcores=NS)

    @functools.partial(
        pl.kernel,
        out_type=jax.ShapeDtypeStruct((NC, NS, ZR, PW), jnp.float32),
        mesh=mesh,
        compiler_params=pltpu.CompilerParams(use_tc_tiling_on_sc=False),
        scratch_types=[
            pltpu.VMEM((CH, K), jnp.int32),
            pltpu.VMEM((CH, K), jnp.int32),
            pltpu.VMEM((K, PW), jnp.float32),
            pltpu.VMEM_SHARED((N, PW), jnp.float32),
        ],
    )
    def _sc_segsum(payload_hbm, src_hbm, dst_hbm, zeros_hbm, out_hbm,
                   src_v, dst_v, rows_v, acc_sh):
        c = jax.lax.axis_index("c")
        s = jax.lax.axis_index("s")
        wid = c * NS + s
        pltpu.sync_copy(zeros_hbm, acc_sh.at[pl.ds(s * ZR, ZR)])
        pltpu.sync_copy(src_hbm.at[wid], src_v)
        pltpu.sync_copy(dst_hbm.at[wid], dst_v)
        plsc.subcore_barrier()

        def body(j, carry):
            pltpu.sync_copy(payload_hbm.at[src_v.at[j]], rows_v)
            pltpu.sync_copy(rows_v, acc_sh.at[dst_v.at[j]], add=True)
            return carry

        jax.lax.fori_loop(0, CH, body, 0)
        plsc.subcore_barrier()
        pltpu.sync_copy(acc_sh.at[pl.ds(s * ZR, ZR)], out_hbm.at[c, s])

    return _sc_segsum


def _stage_c1(acc_ref, p_ref, w2n_ref, w2s_ref, b2_ref,
              o2_ref, mout_ref, stats_ref):
    i = pl.program_id(0)
    p = p_ref[...]
    t = acc_ref[0] + acc_ref[1] + p
    msum = t[:, PLANES:PLANES + 1]
    cnt = t[:, PLANES + 1:PLANES + 2]
    hold2 = (msum > 0).astype(jnp.float32)
    ratio2 = hold2 * cnt / jnp.clip(msum, EPS, None)
    o2 = _dot(t, w2n_ref[...]) + _dot(p, w2s_ref[...])
    o2 = o2 * ratio2 + b2_ref[...] * hold2
    o2_ref[...] = o2
    mout_ref[...] = jnp.clip(msum, 0.0, 1.0)
    _accum_stats(i, o2, stats_ref)


_stage_c1_call = pl.pallas_call(
    _stage_c1,
    grid=(G,),
    in_specs=[pl.BlockSpec((NC, R, PW), lambda i: (0, i, 0)), _rowspec(PW),
              _fullspec(PW, PLANES), _fullspec(PW, PLANES),
              _fullspec(1, PLANES)],
    out_specs=(_rowspec(PLANES), _rowspec(1), _fullspec(2, PLANES)),
    out_shape=(
        jax.ShapeDtypeStruct((N, PLANES), jnp.float32),
        jax.ShapeDtypeStruct((N, 1), jnp.float32),
        jax.ShapeDtypeStruct((2, PLANES), jnp.float32),
    ),
)


def _stage_c2(o2_ref, mout_ref, stats_ref, g2_ref, be2_ref, w3_ref, b3_ref,
              o3_ref, stats3_ref):
    i = pl.program_id(0)
    h2 = _leaky(_norm_from_stats(o2_ref[...], stats_ref[...],
                                 g2_ref[...], be2_ref[...]))
    mout = mout_ref[...]
    hold3 = (mout > 0).astype(jnp.float32)
    ratio3 = hold3 / jnp.clip(mout, EPS, None)
    o3 = _dot(h2 * mout, w3_ref[...]) * ratio3 + b3_ref[...] * hold3
    o3_ref[...] = o3
    _accum_stats(i, o3, stats3_ref)


_stage_c2_call = pl.pallas_call(
    _stage_c2,
    grid=(G,),
    in_specs=[_rowspec(PLANES), _rowspec(1), _fullspec(2, PLANES),
              _fullspec(1, PLANES), _fullspec(1, PLANES),
              _fullspec(PLANES, INPLANES), _fullspec(1, INPLANES)],
    out_specs=(_rowspec(INPLANES), _fullspec(2, INPLANES)),
    out_shape=(
        jax.ShapeDtypeStruct((N, INPLANES), jnp.float32),
        jax.ShapeDtypeStruct((2, INPLANES), jnp.float32),
    ),
)


def _stage_c3(o3_ref, x_ref, m_ref, mout_ref, stats_ref, g3_ref, be3_ref,
              out_ref, omask_ref):
    h3 = _norm_from_stats(o3_ref[...], stats_ref[...],
                          g3_ref[...], be3_ref[...])
    out_ref[...] = _leaky(h3 + x_ref[...])
    omask_ref[...] = jnp.clip(mout_ref[...] + m_ref[...], 0.0, 1.0)


_stage_c3_call = pl.pallas_call(
    _stage_c3,
    grid=(G,),
    in_specs=[_rowspec(INPLANES), _rowspec(INPLANES), _rowspec(1),
              _rowspec(1), _fullspec(2, INPLANES), _fullspec(1, INPLANES),
              _fullspec(1, INPLANES)],
    out_specs=(_rowspec(INPLANES), _rowspec(1)),
    out_shape=(
        jax.ShapeDtypeStruct((N, INPLANES), jnp.float32),
        jax.ShapeDtypeStruct((N, 1), jnp.float32),
    ),
)


def kernel(x, mask, edge_index, W1, b1, g1, be1, W2s, W2n, b2, g2, be2,
           W3, b3, g3, be3):
    src = edge_index[0].reshape(NW, CH, K)
    dst = edge_index[1].reshape(NW, CH, K)
    w1p = jnp.pad(W1, ((0, 0), (0, PW - PLANES)))
    b1p = jnp.pad(b1, (0, PW - PLANES)).reshape(1, PW)
    g1p = jnp.pad(g1, (0, PW - PLANES)).reshape(1, PW)
    be1p = jnp.pad(be1, (0, PW - PLANES)).reshape(1, PW)
    w2np = jnp.pad(W2n, ((0, PW - PLANES), (0, 0)))
    w2sp = jnp.pad(W2s, ((0, PW - PLANES), (0, 0)))

    o1, stats1 = _stage_a1_call(x, mask, w1p, b1p)
    payload = _stage_a2_call(o1, mask, stats1, g1p, be1p)
    zeros_blk = jnp.zeros((ZR, PW), jnp.float32)
    acc = _sc_segsum_call()(payload, src, dst, zeros_blk)
    acc = acc.reshape(NC, N, PW)
    o2, mout, stats2 = _stage_c1_call(acc, payload, w2np, w2sp,
                                      b2.reshape(1, PLANES))
    o3, stats3 = _stage_c2_call(o2, mout, stats2, g2.reshape(1, PLANES),
                                be2.reshape(1, PLANES), W3,
                                b3.reshape(1, INPLANES))
    out, omask = _stage_c3_call(o3, x, mask, mout, stats3,
                                g3.reshape(1, INPLANES),
                                be3.reshape(1, INPLANES))
    return (out, omask)


# speedup vs baseline: 20.6910x; 1.4095x over previous
import functools

import jax
import jax.numpy as jnp
from jax.experimental import pallas as pl
from jax.experimental.pallas import tpu as pltpu
from jax.experimental.pallas import tpu_sc as plsc

N = 10000
E = 320000
INPLANES = 128
PLANES = 32
EXPANSION = 4
EPS = 1e-5

PW = 48
NC = 2
NS = 16
NW = NC * NS
EPW = E // NW
K = 100
CH = EPW // K
NB = 10
ZR = N // NS

G = 10
R = N // G

_HIGH = jax.lax.Precision.HIGHEST


def _dot(a, b):
    return jnp.dot(a, b, preferred_element_type=jnp.float32, precision=_HIGH)


def _leaky(h):
    return jnp.where(h >= 0, h, 0.1 * h)


def _norm_from_stats(o, stats, g, b):
    mu = stats[0:1, :] * (1.0 / N)
    var = stats[1:2, :] * (1.0 / N) - mu * mu
    return g * (o - mu) / jnp.sqrt(var + EPS) + b


def _accum_stats(i, o, stats_ref):
    @pl.when(i == 0)
    def _():
        stats_ref[...] = jnp.zeros_like(stats_ref)
    s = jnp.sum(o, axis=0, keepdims=True)
    sq = jnp.sum(o * o, axis=0, keepdims=True)
    stats_ref[...] += jnp.concatenate([s, sq], axis=0)


def _rowspec(w):
    return pl.BlockSpec((R, w), lambda i: (i, 0))


def _fullspec(h, w):
    return pl.BlockSpec((h, w), lambda i: (0, 0))


def _stage_a1(x_ref, m_ref, w1_ref, b1_ref, o_ref, stats_ref):
    i = pl.program_id(0)
    m = m_ref[...]
    hold = (m > 0).astype(jnp.float32)
    ratio = hold / jnp.clip(m, EPS, None)
    o = _dot(x_ref[...] * m, w1_ref[...])
    o = o * ratio + b1_ref[...] * hold
    o_ref[...] = o
    _accum_stats(i, o, stats_ref)


_stage_a1_call = pl.pallas_call(
    _stage_a1,
    grid=(G,),
    in_specs=[_rowspec(INPLANES), _rowspec(1), _fullspec(INPLANES, PW),
              _fullspec(1, PW)],
    out_specs=(_rowspec(PW), _fullspec(2, PW)),
    out_shape=(
        jax.ShapeDtypeStruct((N, PW), jnp.float32),
        jax.ShapeDtypeStruct((2, PW), jnp.float32),
    ),
)


def _stage_a2(o_ref, m_ref, stats_ref, g1_ref, be1_ref, pay_ref):
    m = m_ref[...]
    hold = (m > 0).astype(jnp.float32)
    h = _leaky(_norm_from_stats(o_ref[...], stats_ref[...],
                                g1_ref[...], be1_ref[...]))
    hm = h * hold
    ci = jax.lax.broadcasted_iota(jnp.int32, (R, PW), 1)
    holdb = jnp.broadcast_to(hold, (R, PW))
    pay_ref[...] = jnp.where(ci < PLANES, hm,
                             jnp.where(ci == PLANES, holdb,
                                       jnp.where(ci == PLANES + 1, 1.0, 0.0)))


_stage_a2_call = pl.pallas_call(
    _stage_a2,
    grid=(G,),
    in_specs=[_rowspec(PW), _rowspec(1), _fullspec(2, PW), _fullspec(1, PW),
              _fullspec(1, PW)],
    out_specs=_rowspec(PW),
    out_shape=jax.ShapeDtypeStruct((N, PW), jnp.float32),
)


@functools.cache
def _sc_segsum_call():
    mesh = plsc.VectorSubcoreMesh(
        core_axis_name="c", subcore_axis_name="s",
        num_cores=NC, num_subcores=NS)

    @functools.partial(
        pl.kernel,
        out_type=jax.ShapeDtypeStruct((NC, NS, ZR, PW), jnp.float32),
        mesh=mesh,
        compiler_params=pltpu.CompilerParams(use_tc_tiling_on_sc=False),
        scratch_types=[
            pltpu.VMEM((CH, K), jnp.int32),
            pltpu.VMEM((CH, K), jnp.int32),
            pltpu.VMEM((NB, K, PW), jnp.float32),
            pltpu.VMEM_SHARED((N, PW), jnp.float32),
            pltpu.SemaphoreType.DMA((NB,)),
            pltpu.SemaphoreType.DMA((NB,)),
        ],
    )
    def _sc_segsum(payload_hbm, src_hbm, dst_hbm, zeros_hbm, out_hbm,
                   src_v, dst_v, bufs, acc_sh, gsem, ssem):
        c = jax.lax.axis_index("c")
        s = jax.lax.axis_index("s")
        wid = c * NS + s
        pltpu.sync_copy(zeros_hbm, acc_sh.at[pl.ds(s * ZR, ZR)])
        pltpu.sync_copy(src_hbm.at[wid], src_v)
        pltpu.sync_copy(dst_hbm.at[wid], dst_v)
        plsc.subcore_barrier()

        def group(g, carry):
            base = g * NB
            for b in range(NB):
                pltpu.async_copy(payload_hbm.at[src_v.at[base + b]],
                                 bufs.at[b], gsem.at[b])
            for b in range(NB):
                pltpu.make_async_copy(payload_hbm.at[src_v.at[base + b]],
                                      bufs.at[b], gsem.at[b]).wait()
                pltpu.async_copy(bufs.at[b], acc_sh.at[dst_v.at[base + b]],
                                 ssem.at[b], add=True)
            for b in range(NB):
                pltpu.make_async_copy(bufs.at[b],
                                      acc_sh.at[dst_v.at[base + b]],
                                      ssem.at[b]).wait()
            return carry

        jax.lax.fori_loop(0, CH // NB, group, 0)
        plsc.subcore_barrier()
        pltpu.sync_copy(acc_sh.at[pl.ds(s * ZR, ZR)], out_hbm.at[c, s])

    return _sc_segsum


def _stage_c1(acc_ref, p_ref, w2n_ref, w2s_ref, b2_ref,
              o2_ref, mout_ref, stats_ref):
    i = pl.program_id(0)
    p = p_ref[...]
    t = acc_ref[0] + acc_ref[1] + p
    msum = t[:, PLANES:PLANES + 1]
    cnt = t[:, PLANES + 1:PLANES + 2]
    hold2 = (msum > 0).astype(jnp.float32)
    ratio2 = hold2 * cnt / jnp.clip(msum, EPS, None)
    o2 = _dot(t, w2n_ref[...]) + _dot(p, w2s_ref[...])
    o2 = o2 * ratio2 + b2_ref[...] * hold2
    o2_ref[...] = o2
    mout_ref[...] = jnp.clip(msum, 0.0, 1.0)
    _accum_stats(i, o2, stats_ref)


_stage_c1_call = pl.pallas_call(
    _stage_c1,
    grid=(G,),
    in_specs=[pl.BlockSpec((NC, R, PW), lambda i: (0, i, 0)), _rowspec(PW),
              _fullspec(PW, PLANES), _fullspec(PW, PLANES),
              _fullspec(1, PLANES)],
    out_specs=(_rowspec(PLANES), _rowspec(1), _fullspec(2, PLANES)),
    out_shape=(
        jax.ShapeDtypeStruct((N, PLANES), jnp.float32),
        jax.ShapeDtypeStruct((N, 1), jnp.float32),
        jax.ShapeDtypeStruct((2, PLANES), jnp.float32),
    ),
)


def _stage_c2(o2_ref, mout_ref, stats_ref, g2_ref, be2_ref, w3_ref, b3_ref,
              o3_ref, stats3_ref):
    i = pl.program_id(0)
    h2 = _leaky(_norm_from_stats(o2_ref[...], stats_ref[...],
                                 g2_ref[...], be2_ref[...]))
    mout = mout_ref[...]
    hold3 = (mout > 0).astype(jnp.float32)
    ratio3 = hold3 / jnp.clip(mout, EPS, None)
    o3 = _dot(h2 * mout, w3_ref[...]) * ratio3 + b3_ref[...] * hold3
    o3_ref[...] = o3
    _accum_stats(i, o3, stats3_ref)


_stage_c2_call = pl.pallas_call(
    _stage_c2,
    grid=(G,),
    in_specs=[_rowspec(PLANES), _rowspec(1), _fullspec(2, PLANES),
              _fullspec(1, PLANES), _fullspec(1, PLANES),
              _fullspec(PLANES, INPLANES), _fullspec(1, INPLANES)],
    out_specs=(_rowspec(INPLANES), _fullspec(2, INPLANES)),
    out_shape=(
        jax.ShapeDtypeStruct((N, INPLANES), jnp.float32),
        jax.ShapeDtypeStruct((2, INPLANES), jnp.float32),
    ),
)


def _stage_c3(o3_ref, x_ref, m_ref, mout_ref, stats_ref, g3_ref, be3_ref,
              out_ref, omask_ref):
    h3 = _norm_from_stats(o3_ref[...], stats_ref[...],
                          g3_ref[...], be3_ref[...])
    out_ref[...] = _leaky(h3 + x_ref[...])
    omask_ref[...] = jnp.clip(mout_ref[...] + m_ref[...], 0.0, 1.0)


_stage_c3_call = pl.pallas_call(
    _stage_c3,
    grid=(G,),
    in_specs=[_rowspec(INPLANES), _rowspec(INPLANES), _rowspec(1),
              _rowspec(1), _fullspec(2, INPLANES), _fullspec(1, INPLANES),
              _fullspec(1, INPLANES)],
    out_specs=(_rowspec(INPLANES), _rowspec(1)),
    out_shape=(
        jax.ShapeDtypeStruct((N, INPLANES), jnp.float32),
        jax.ShapeDtypeStruct((N, 1), jnp.float32),
    ),
)


def kernel(x, mask, edge_index, W1, b1, g1, be1, W2s, W2n, b2, g2, be2,
           W3, b3, g3, be3):
    src = edge_index[0].reshape(NW, CH, K)
    dst = edge_index[1].reshape(NW, CH, K)
    w1p = jnp.pad(W1, ((0, 0), (0, PW - PLANES)))
    b1p = jnp.pad(b1, (0, PW - PLANES)).reshape(1, PW)
    g1p = jnp.pad(g1, (0, PW - PLANES)).reshape(1, PW)
    be1p = jnp.pad(be1, (0, PW - PLANES)).reshape(1, PW)
    w2np = jnp.pad(W2n, ((0, PW - PLANES), (0, 0)))
    w2sp = jnp.pad(W2s, ((0, PW - PLANES), (0, 0)))

    o1, stats1 = _stage_a1_call(x, mask, w1p, b1p)
    payload = _stage_a2_call(o1, mask, stats1, g1p, be1p)
    zeros_blk = jnp.zeros((ZR, PW), jnp.float32)
    acc = _sc_segsum_call()(payload, src, dst, zeros_blk)
    acc = acc.reshape(NC, N, PW)
    o2, mout, stats2 = _stage_c1_call(acc, payload, w2np, w2sp,
                                      b2.reshape(1, PLANES))
    o3, stats3 = _stage_c2_call(o2, mout, stats2, g2.reshape(1, PLANES),
                                be2.reshape(1, PLANES), W3,
                                b3.reshape(1, INPLANES))
    out, omask = _stage_c3_call(o3, x, mask, mout, stats3,
                                g3.reshape(1, INPLANES),
                                be3.reshape(1, INPLANES))
    return (out, omask)


# speedup vs baseline: 22.3905x; 1.0821x over previous
import functools

import jax
import jax.numpy as jnp
from jax.experimental import pallas as pl
from jax.experimental.pallas import tpu as pltpu
from jax.experimental.pallas import tpu_sc as plsc

N = 10000
E = 320000
INPLANES = 128
PLANES = 32
EXPANSION = 4
EPS = 1e-5

PW = 48
NC = 2
NS = 16
NW = NC * NS
EPW = E // NW
K = 100
CH = EPW // K
NB = 10
ZR = N // NS

G = 10
R = N // G

_HIGH = jax.lax.Precision.DEFAULT


def _dot(a, b):
    return jnp.dot(a, b, preferred_element_type=jnp.float32, precision=_HIGH)


def _leaky(h):
    return jnp.where(h >= 0, h, 0.1 * h)


def _norm_from_stats(o, stats, g, b):
    mu = stats[0:1, :] * (1.0 / N)
    var = stats[1:2, :] * (1.0 / N) - mu * mu
    return g * (o - mu) / jnp.sqrt(var + EPS) + b


def _accum_stats(i, o, stats_ref):
    @pl.when(i == 0)
    def _():
        stats_ref[...] = jnp.zeros_like(stats_ref)
    s = jnp.sum(o, axis=0, keepdims=True)
    sq = jnp.sum(o * o, axis=0, keepdims=True)
    stats_ref[...] += jnp.concatenate([s, sq], axis=0)


def _rowspec(w):
    return pl.BlockSpec((R, w), lambda i: (i, 0))


def _fullspec(h, w):
    return pl.BlockSpec((h, w), lambda i: (0, 0))


def _stage_a1(x_ref, m_ref, w1_ref, b1_ref, o_ref, stats_ref):
    i = pl.program_id(0)
    m = m_ref[...]
    hold = (m > 0).astype(jnp.float32)
    ratio = hold / jnp.clip(m, EPS, None)
    o = _dot(x_ref[...] * m, w1_ref[...])
    o = o * ratio + b1_ref[...] * hold
    o_ref[...] = o
    _accum_stats(i, o, stats_ref)


_stage_a1_call = pl.pallas_call(
    _stage_a1,
    grid=(G,),
    in_specs=[_rowspec(INPLANES), _rowspec(1), _fullspec(INPLANES, PW),
              _fullspec(1, PW)],
    out_specs=(_rowspec(PW), _fullspec(2, PW)),
    out_shape=(
        jax.ShapeDtypeStruct((N, PW), jnp.float32),
        jax.ShapeDtypeStruct((2, PW), jnp.float32),
    ),
)


def _stage_a2(o_ref, m_ref, stats_ref, g1_ref, be1_ref, pay_ref):
    m = m_ref[...]
    hold = (m > 0).astype(jnp.float32)
    h = _leaky(_norm_from_stats(o_ref[...], stats_ref[...],
                                g1_ref[...], be1_ref[...]))
    hm = h * hold
    ci = jax.lax.broadcasted_iota(jnp.int32, (R, PW), 1)
    holdb = jnp.broadcast_to(hold, (R, PW))
    pay_ref[...] = jnp.where(ci < PLANES, hm,
                             jnp.where(ci == PLANES, holdb,
                                       jnp.where(ci == PLANES + 1, 1.0, 0.0)))


_stage_a2_call = pl.pallas_call(
    _stage_a2,
    grid=(G,),
    in_specs=[_rowspec(PW), _rowspec(1), _fullspec(2, PW), _fullspec(1, PW),
              _fullspec(1, PW)],
    out_specs=_rowspec(PW),
    out_shape=jax.ShapeDtypeStruct((N, PW), jnp.float32),
)


@functools.cache
def _sc_segsum_call():
    mesh = plsc.VectorSubcoreMesh(
        core_axis_name="c", subcore_axis_name="s",
        num_cores=NC, num_subcores=NS)

    @functools.partial(
        pl.kernel,
        out_type=jax.ShapeDtypeStruct((NC, NS, ZR, PW), jnp.float32),
        mesh=mesh,
        compiler_params=pltpu.CompilerParams(use_tc_tiling_on_sc=False),
        scratch_types=[
            pltpu.VMEM((CH, K), jnp.int32),
            pltpu.VMEM((CH, K), jnp.int32),
            pltpu.VMEM((NB, K, PW), jnp.float32),
            pltpu.VMEM_SHARED((N, PW), jnp.float32),
            pltpu.SemaphoreType.DMA((NB,)),
            pltpu.SemaphoreType.DMA((NB,)),
        ],
    )
    def _sc_segsum(payload_hbm, src_hbm, dst_hbm, zeros_hbm, out_hbm,
                   src_v, dst_v, bufs, acc_sh, gsem, ssem):
        c = jax.lax.axis_index("c")
        s = jax.lax.axis_index("s")
        wid = c * NS + s
        pltpu.sync_copy(zeros_hbm, acc_sh.at[pl.ds(s * ZR, ZR)])
        pltpu.sync_copy(src_hbm.at[wid], src_v)
        pltpu.sync_copy(dst_hbm.at[wid], dst_v)
        plsc.subcore_barrier()

        def group(g, carry):
            base = g * NB
            for b in range(NB):
                pltpu.async_copy(payload_hbm.at[src_v.at[base + b]],
                                 bufs.at[b], gsem.at[b])
            for b in range(NB):
                pltpu.make_async_copy(payload_hbm.at[src_v.at[base + b]],
                                      bufs.at[b], gsem.at[b]).wait()
                pltpu.async_copy(bufs.at[b], acc_sh.at[dst_v.at[base + b]],
                                 ssem.at[b], add=True)
            for b in range(NB):
                pltpu.make_async_copy(bufs.at[b],
                                      acc_sh.at[dst_v.at[base + b]],
                                      ssem.at[b]).wait()
            return carry

        jax.lax.fori_loop(0, CH // NB, group, 0)
        plsc.subcore_barrier()
        pltpu.sync_copy(acc_sh.at[pl.ds(s * ZR, ZR)], out_hbm.at[c, s])

    return _sc_segsum


def _stage_c1(acc_ref, p_ref, w2n_ref, w2s_ref, b2_ref,
              o2_ref, mout_ref, stats_ref):
    i = pl.program_id(0)
    p = p_ref[...]
    t = acc_ref[0] + acc_ref[1] + p
    msum = t[:, PLANES:PLANES + 1]
    cnt = t[:, PLANES + 1:PLANES + 2]
    hold2 = (msum > 0).astype(jnp.float32)
    ratio2 = hold2 * cnt / jnp.clip(msum, EPS, None)
    o2 = _dot(t, w2n_ref[...]) + _dot(p, w2s_ref[...])
    o2 = o2 * ratio2 + b2_ref[...] * hold2
    o2_ref[...] = o2
    mout_ref[...] = jnp.clip(msum, 0.0, 1.0)
    _accum_stats(i, o2, stats_ref)


_stage_c1_call = pl.pallas_call(
    _stage_c1,
    grid=(G,),
    in_specs=[pl.BlockSpec((NC, R, PW), lambda i: (0, i, 0)), _rowspec(PW),
              _fullspec(PW, PLANES), _fullspec(PW, PLANES),
              _fullspec(1, PLANES)],
    out_specs=(_rowspec(PLANES), _rowspec(1), _fullspec(2, PLANES)),
    out_shape=(
        jax.ShapeDtypeStruct((N, PLANES), jnp.float32),
        jax.ShapeDtypeStruct((N, 1), jnp.float32),
        jax.ShapeDtypeStruct((2, PLANES), jnp.float32),
    ),
)


def _stage_c2(o2_ref, mout_ref, stats_ref, g2_ref, be2_ref, w3_ref, b3_ref,
              o3_ref, stats3_ref):
    i = pl.program_id(0)
    h2 = _leaky(_norm_from_stats(o2_ref[...], stats_ref[...],
                                 g2_ref[...], be2_ref[...]))
    mout = mout_ref[...]
    hold3 = (mout > 0).astype(jnp.float32)
    ratio3 = hold3 / jnp.clip(mout, EPS, None)
    o3 = _dot(h2 * mout, w3_ref[...]) * ratio3 + b3_ref[...] * hold3
    o3_ref[...] = o3
    _accum_stats(i, o3, stats3_ref)


_stage_c2_call = pl.pallas_call(
    _stage_c2,
    grid=(G,),
    in_specs=[_rowspec(PLANES), _rowspec(1), _fullspec(2, PLANES),
              _fullspec(1, PLANES), _fullspec(1, PLANES),
              _fullspec(PLANES, INPLANES), _fullspec(1, INPLANES)],
    out_specs=(_rowspec(INPLANES), _fullspec(2, INPLANES)),
    out_shape=(
        jax.ShapeDtypeStruct((N, INPLANES), jnp.float32),
        jax.ShapeDtypeStruct((2, INPLANES), jnp.float32),
    ),
)


def _stage_c3(o3_ref, x_ref, m_ref, mout_ref, stats_ref, g3_ref, be3_ref,
              out_ref, omask_ref):
    h3 = _norm_from_stats(o3_ref[...], stats_ref[...],
                          g3_ref[...], be3_ref[...])
    out_ref[...] = _leaky(h3 + x_ref[...])
    omask_ref[...] = jnp.clip(mout_ref[...] + m_ref[...], 0.0, 1.0)


_stage_c3_call = pl.pallas_call(
    _stage_c3,
    grid=(G,),
    in_specs=[_rowspec(INPLANES), _rowspec(INPLANES), _rowspec(1),
              _rowspec(1), _fullspec(2, INPLANES), _fullspec(1, INPLANES),
              _fullspec(1, INPLANES)],
    out_specs=(_rowspec(INPLANES), _rowspec(1)),
    out_shape=(
        jax.ShapeDtypeStruct((N, INPLANES), jnp.float32),
        jax.ShapeDtypeStruct((N, 1), jnp.float32),
    ),
)


def kernel(x, mask, edge_index, W1, b1, g1, be1, W2s, W2n, b2, g2, be2,
           W3, b3, g3, be3):
    src = edge_index[0].reshape(NW, CH, K)
    dst = edge_index[1].reshape(NW, CH, K)
    w1p = jnp.pad(W1, ((0, 0), (0, PW - PLANES)))
    b1p = jnp.pad(b1, (0, PW - PLANES)).reshape(1, PW)
    g1p = jnp.pad(g1, (0, PW - PLANES)).reshape(1, PW)
    be1p = jnp.pad(be1, (0, PW - PLANES)).reshape(1, PW)
    w2np = jnp.pad(W2n, ((0, PW - PLANES), (0, 0)))
    w2sp = jnp.pad(W2s, ((0, PW - PLANES), (0, 0)))

    o1, stats1 = _stage_a1_call(x, mask, w1p, b1p)
    payload = _stage_a2_call(o1, mask, stats1, g1p, be1p)
    zeros_blk = jnp.zeros((ZR, PW), jnp.float32)
    acc = _sc_segsum_call()(payload, src, dst, zeros_blk)
    acc = acc.reshape(NC, N, PW)
    o2, mout, stats2 = _stage_c1_call(acc, payload, w2np, w2sp,
                                      b2.reshape(1, PLANES))
    o3, stats3 = _stage_c2_call(o2, mout, stats2, g2.reshape(1, PLANES),
                                be2.reshape(1, PLANES), W3,
                                b3.reshape(1, INPLANES))
    out, omask = _stage_c3_call(o3, x, mask, mout, stats3,
                                g3.reshape(1, INPLANES),
                                be3.reshape(1, INPLANES))
    return (out, omask)


# speedup vs baseline: 23.5400x; 1.0513x over previous
import functools

import jax
import jax.numpy as jnp
from jax.experimental import pallas as pl
from jax.experimental.pallas import tpu as pltpu
from jax.experimental.pallas import tpu_sc as plsc

N = 10000
E = 320000
INPLANES = 128
PLANES = 32
EXPANSION = 4
EPS = 1e-5

PW = 48
NC = 2
NS = 16
NW = NC * NS
EPW = E // NW
K = 100
CH = EPW // K
NB = 10
ZR = N // NS

G = 5
R = N // G

_HIGH = jax.lax.Precision.DEFAULT


def _dot(a, b):
    return jnp.dot(a, b, preferred_element_type=jnp.float32, precision=_HIGH)


def _leaky(h):
    return jnp.where(h >= 0, h, 0.1 * h)


def _norm_from_stats(o, stats, g, b):
    mu = stats[0:1, :] * (1.0 / N)
    var = stats[1:2, :] * (1.0 / N) - mu * mu
    return g * (o - mu) / jnp.sqrt(var + EPS) + b


def _accum_stats(i, o, stats_ref):
    @pl.when(i == 0)
    def _():
        stats_ref[...] = jnp.zeros_like(stats_ref)
    s = jnp.sum(o, axis=0, keepdims=True)
    sq = jnp.sum(o * o, axis=0, keepdims=True)
    stats_ref[...] += jnp.concatenate([s, sq], axis=0)


def _rowspec(w):
    return pl.BlockSpec((R, w), lambda i: (i, 0))


def _fullspec(h, w):
    return pl.BlockSpec((h, w), lambda i: (0, 0))


def _stage_a1(x_ref, m_ref, w1_ref, b1_ref, o_ref, stats_ref):
    i = pl.program_id(0)
    m = m_ref[...]
    hold = (m > 0).astype(jnp.float32)
    ratio = hold / jnp.clip(m, EPS, None)
    o = _dot(x_ref[...] * m, w1_ref[...])
    o = o * ratio + b1_ref[...] * hold
    o_ref[...] = o
    _accum_stats(i, o, stats_ref)


_stage_a1_call = pl.pallas_call(
    _stage_a1,
    grid=(G,),
    in_specs=[_rowspec(INPLANES), _rowspec(1), _fullspec(INPLANES, PW),
              _fullspec(1, PW)],
    out_specs=(_rowspec(PW), _fullspec(2, PW)),
    out_shape=(
        jax.ShapeDtypeStruct((N, PW), jnp.float32),
        jax.ShapeDtypeStruct((2, PW), jnp.float32),
    ),
)


def _stage_a2(o_ref, m_ref, stats_ref, g1_ref, be1_ref, pay_ref):
    m = m_ref[...]
    hold = (m > 0).astype(jnp.float32)
    h = _leaky(_norm_from_stats(o_ref[...], stats_ref[...],
                                g1_ref[...], be1_ref[...]))
    hm = h * hold
    ci = jax.lax.broadcasted_iota(jnp.int32, (R, PW), 1)
    holdb = jnp.broadcast_to(hold, (R, PW))
    pay_ref[...] = jnp.where(ci < PLANES, hm,
                             jnp.where(ci == PLANES, holdb,
                                       jnp.where(ci == PLANES + 1, 1.0, 0.0)))


_stage_a2_call = pl.pallas_call(
    _stage_a2,
    grid=(G,),
    in_specs=[_rowspec(PW), _rowspec(1), _fullspec(2, PW), _fullspec(1, PW),
              _fullspec(1, PW)],
    out_specs=_rowspec(PW),
    out_shape=jax.ShapeDtypeStruct((N, PW), jnp.float32),
)


@functools.cache
def _sc_segsum_call():
    mesh = plsc.VectorSubcoreMesh(
        core_axis_name="c", subcore_axis_name="s",
        num_cores=NC, num_subcores=NS)

    @functools.partial(
        pl.kernel,
        out_type=jax.ShapeDtypeStruct((NC, NS, ZR, PW), jnp.float32),
        mesh=mesh,
        compiler_params=pltpu.CompilerParams(use_tc_tiling_on_sc=False),
        scratch_types=[
            pltpu.VMEM((CH, K), jnp.int32),
            pltpu.VMEM((CH, K), jnp.int32),
            pltpu.VMEM((NB, K, PW), jnp.float32),
            pltpu.VMEM_SHARED((N, PW), jnp.float32),
            pltpu.SemaphoreType.DMA((NB,)),
            pltpu.SemaphoreType.DMA((NB,)),
        ],
    )
    def _sc_segsum(payload_hbm, src_hbm, dst_hbm, zeros_hbm, out_hbm,
                   src_v, dst_v, bufs, acc_sh, gsem, ssem):
        c = jax.lax.axis_index("c")
        s = jax.lax.axis_index("s")
        wid = c * NS + s
        pltpu.sync_copy(zeros_hbm, acc_sh.at[pl.ds(s * ZR, ZR)])
        pltpu.sync_copy(src_hbm.at[wid], src_v)
        pltpu.sync_copy(dst_hbm.at[wid], dst_v)
        plsc.subcore_barrier()

        def group(g, carry):
            base = g * NB
            for b in range(NB):
                pltpu.async_copy(payload_hbm.at[src_v.at[base + b]],
                                 bufs.at[b], gsem.at[b])
            for b in range(NB):
                pltpu.make_async_copy(payload_hbm.at[src_v.at[base + b]],
                                      bufs.at[b], gsem.at[b]).wait()
                pltpu.async_copy(bufs.at[b], acc_sh.at[dst_v.at[base + b]],
                                 ssem.at[b], add=True)
            for b in range(NB):
                pltpu.make_async_copy(bufs.at[b],
                                      acc_sh.at[dst_v.at[base + b]],
                                      ssem.at[b]).wait()
            return carry

        jax.lax.fori_loop(0, CH // NB, group, 0)
        plsc.subcore_barrier()
        pltpu.sync_copy(acc_sh.at[pl.ds(s * ZR, ZR)], out_hbm.at[c, s])

    return _sc_segsum


def _stage_c1(acc_ref, p_ref, w2n_ref, w2s_ref, b2_ref,
              o2_ref, mout_ref, stats_ref):
    i = pl.program_id(0)
    p = p_ref[...]
    t = acc_ref[0] + acc_ref[1] + p
    msum = t[:, PLANES:PLANES + 1]
    cnt = t[:, PLANES + 1:PLANES + 2]
    hold2 = (msum > 0).astype(jnp.float32)
    ratio2 = hold2 * cnt / jnp.clip(msum, EPS, None)
    o2 = _dot(t, w2n_ref[...]) + _dot(p, w2s_ref[...])
    o2 = o2 * ratio2 + b2_ref[...] * hold2
    o2_ref[...] = o2
    mout_ref[...] = jnp.clip(msum, 0.0, 1.0)
    _accum_stats(i, o2, stats_ref)


_stage_c1_call = pl.pallas_call(
    _stage_c1,
    grid=(G,),
    in_specs=[pl.BlockSpec((NC, R, PW), lambda i: (0, i, 0)), _rowspec(PW),
              _fullspec(PW, PLANES), _fullspec(PW, PLANES),
              _fullspec(1, PLANES)],
    out_specs=(_rowspec(PLANES), _rowspec(1), _fullspec(2, PLANES)),
    out_shape=(
        jax.ShapeDtypeStruct((N, PLANES), jnp.float32),
        jax.ShapeDtypeStruct((N, 1), jnp.float32),
        jax.ShapeDtypeStruct((2, PLANES), jnp.float32),
    ),
)


def _stage_c2(o2_ref, mout_ref, stats_ref, g2_ref, be2_ref, w3_ref, b3_ref,
              o3_ref, stats3_ref):
    i = pl.program_id(0)
    h2 = _leaky(_norm_from_stats(o2_ref[...], stats_ref[...],
                                 g2_ref[...], be2_ref[...]))
    mout = mout_ref[...]
    hold3 = (mout > 0).astype(jnp.float32)
    ratio3 = hold3 / jnp.clip(mout, EPS, None)
    o3 = _dot(h2 * mout, w3_ref[...]) * ratio3 + b3_ref[...] * hold3
    o3_ref[...] = o3
    _accum_stats(i, o3, stats3_ref)


_stage_c2_call = pl.pallas_call(
    _stage_c2,
    grid=(G,),
    in_specs=[_rowspec(PLANES), _rowspec(1), _fullspec(2, PLANES),
              _fullspec(1, PLANES), _fullspec(1, PLANES),
              _fullspec(PLANES, INPLANES), _fullspec(1, INPLANES)],
    out_specs=(_rowspec(INPLANES), _fullspec(2, INPLANES)),
    out_shape=(
        jax.ShapeDtypeStruct((N, INPLANES), jnp.float32),
        jax.ShapeDtypeStruct((2, INPLANES), jnp.float32),
    ),
)


def _stage_c3(o3_ref, x_ref, m_ref, mout_ref, stats_ref, g3_ref, be3_ref,
              out_ref, omask_ref):
    h3 = _norm_from_stats(o3_ref[...], stats_ref[...],
                          g3_ref[...], be3_ref[...])
    out_ref[...] = _leaky(h3 + x_ref[...])
    omask_ref[...] = jnp.clip(mout_ref[...] + m_ref[...], 0.0, 1.0)


_stage_c3_call = pl.pallas_call(
    _stage_c3,
    grid=(G,),
    in_specs=[_rowspec(INPLANES), _rowspec(INPLANES), _rowspec(1),
              _rowspec(1), _fullspec(2, INPLANES), _fullspec(1, INPLANES),
              _fullspec(1, INPLANES)],
    out_specs=(_rowspec(INPLANES), _rowspec(1)),
    out_shape=(
        jax.ShapeDtypeStruct((N, INPLANES), jnp.float32),
        jax.ShapeDtypeStruct((N, 1), jnp.float32),
    ),
)


def kernel(x, mask, edge_index, W1, b1, g1, be1, W2s, W2n, b2, g2, be2,
           W3, b3, g3, be3):
    src = edge_index[0].reshape(NW, CH, K)
    dst = edge_index[1].reshape(NW, CH, K)
    w1p = jnp.pad(W1, ((0, 0), (0, PW - PLANES)))
    b1p = jnp.pad(b1, (0, PW - PLANES)).reshape(1, PW)
    g1p = jnp.pad(g1, (0, PW - PLANES)).reshape(1, PW)
    be1p = jnp.pad(be1, (0, PW - PLANES)).reshape(1, PW)
    w2np = jnp.pad(W2n, ((0, PW - PLANES), (0, 0)))
    w2sp = jnp.pad(W2s, ((0, PW - PLANES), (0, 0)))

    o1, stats1 = _stage_a1_call(x, mask, w1p, b1p)
    payload = _stage_a2_call(o1, mask, stats1, g1p, be1p)
    zeros_blk = jnp.zeros((ZR, PW), jnp.float32)
    acc = _sc_segsum_call()(payload, src, dst, zeros_blk)
    acc = acc.reshape(NC, N, PW)
    o2, mout, stats2 = _stage_c1_call(acc, payload, w2np, w2sp,
                                      b2.reshape(1, PLANES))
    o3, stats3 = _stage_c2_call(o2, mout, stats2, g2.reshape(1, PLANES),
                                be2.reshape(1, PLANES), W3,
                                b3.reshape(1, INPLANES))
    out, omask = _stage_c3_call(o3, x, mask, mout, stats3,
                                g3.reshape(1, INPLANES),
                                be3.reshape(1, INPLANES))
    return (out, omask)
